# Initial kernel scaffold; baseline (speedup 1.0000x reference)
import jax
import jax.numpy as jnp
from jax.experimental import pallas as pl


def kernel(x, edge_index, h, c, W, b, Wsrc, Wdst, att_src, att_dst, gat_bias):
    raise NotImplementedError("write your pallas kernel here")


# speedup vs baseline: 1.4095x; 1.4095x over previous
import jax
import jax.numpy as jnp
from jax.experimental import pallas as pl
from jax.experimental.pallas import tpu as pltpu

N = 10000
D = 128


def _lstm_body(gi, gf, gc, go, c_ref, h_out, c_out):
    i = jax.nn.sigmoid(gi[...])
    f = jax.nn.sigmoid(gf[...])
    t = jnp.tanh(gc[...])
    c_new = f * c_ref[...] + i * t
    o = jax.nn.sigmoid(go[...])
    h_out[...] = o * jnp.tanh(c_new)
    c_out[...] = c_new


def kernel(x, edge_index, h, c, W, b, Wsrc, Wdst, att_src, att_dst, gat_bias):
    src = edge_index[0]
    dst = edge_index[1]

    xs = jnp.einsum("nd,gde->gne", h, Wsrc)
    asrc = jnp.einsum("gne,ge->gn", xs, att_src)
    adst = jnp.einsum("nd,gde,ge->gn", h, Wdst, att_dst)
    lin = jnp.einsum("nd,gde->gne", x, W) + (b + gat_bias)[:, None, :]

    gates = []
    for g in range(4):
        e = jax.nn.leaky_relu(asrc[g][src] + adst[g][dst], 0.2)
        p = jnp.exp(e)
        denom = jax.ops.segment_sum(p, dst, num_segments=N)
        alpha = p / (denom[dst] + 1e-16)
        out = jax.ops.segment_sum(xs[g][src] * alpha[:, None], dst, num_segments=N)
        gates.append(lin[g] + out)

    h_new, c_new = pl.pallas_call(
        _lstm_body,
        out_shape=(
            jax.ShapeDtypeStruct((N, D), jnp.float32),
            jax.ShapeDtypeStruct((N, D), jnp.float32),
        ),
        grid=(5,),
        in_specs=[pl.BlockSpec((2000, D), lambda i: (i, 0))] * 5,
        out_specs=(
            pl.BlockSpec((2000, D), lambda i: (i, 0)),
            pl.BlockSpec((2000, D), lambda i: (i, 0)),
        ),
    )(gates[0], gates[1], gates[2], gates[3], c)
    return h_new, c_new


# speedup vs baseline: 10.7267x; 7.6101x over previous
import functools

import jax
import jax.numpy as jnp
from jax import lax
from jax.experimental import pallas as pl
from jax.experimental.pallas import tpu as pltpu
from jax.experimental.pallas import tpu_sc as plsc

N = 10000
NP = 10240
D = 128
E = 320000
NT = 16
EPT = E // NT
NITER = 157
EPAD = NITER * 128
NCH = EPAD // 16
ROWS_PER_TILE = NP // NT
ACC_ROWS = 4096
PASSES = ((0, 4096), (4096, 4096), (8192, 2048))


def _phase1_body(x_ref, h_ref, W_ref, b_ref, Wsrc_ref, Wdst_ref, asw_ref,
                 adw_ref, gb_ref, xs_ref, asrc_ref, adst_ref, lin_ref):
    xb = x_ref[...]
    hb = h_ref[...]
    for g in range(4):
        xsg = jnp.dot(hb, Wsrc_ref[g], preferred_element_type=jnp.float32)
        xs_ref[g] = xsg
        asrc_ref[g] = jnp.sum(xsg * asw_ref[g][None, :], axis=1)
        wv = jnp.sum(Wdst_ref[g] * adw_ref[g][None, :], axis=1)
        adst_ref[g] = jnp.sum(hb * wv[None, :], axis=1)
        lin_ref[g] = (jnp.dot(xb, W_ref[g], preferred_element_type=jnp.float32)
                      + (b_ref[g] + gb_ref[g])[None, :])


def _phase3_body(lin_ref, conv_ref, c_ref, h_out, c_out):
    gates = lin_ref[...] + conv_ref[...]
    ig = jax.nn.sigmoid(gates[0])
    fg = jax.nn.sigmoid(gates[1])
    tg = jnp.tanh(gates[2])
    c_new = fg * c_ref[...] + ig * tg
    og = jax.nn.sigmoid(gates[3])
    h_out[...] = og * jnp.tanh(c_new)
    c_out[...] = c_new


def _sc_body(xs_hbm, asrc_hbm, adst_hbm, srce_hbm, dste_hbm, out_hbm,
             src_idx, dst_v, p_alpha, big_v, denom_v, idx80_v,
             scidx_v, acc_s, denom_s, sem):
    c = lax.axis_index("c")
    s = lax.axis_index("s")
    ebase = s * EPT

    zero16 = jnp.zeros((16,), jnp.float32)
    zero16i = jnp.zeros((16,), jnp.int32)
    for q in range(NP // 128 // 16):
        idx80_v[pl.ds(q * 16, 16)] = lax.iota(jnp.int32, 16) + (q * 16)

    pltpu.sync_copy(srce_hbm.at[pl.ds(ebase, EPT)], src_idx.at[pl.ds(0, EPT)])
    pltpu.sync_copy(dste_hbm.at[pl.ds(ebase, EPT)], dst_v.at[pl.ds(0, EPT)])
    for i in range((EPAD - EPT) // 16):
        src_idx[pl.ds(EPT + i * 16, 16)] = zero16i
        dst_v[pl.ds(EPT + i * 16, 16)] = zero16i

    for gg in range(2):
        g = 2 * c + gg

        off = (c * (2 * NP)).astype(jnp.int32) if gg == 0 else jnp.int32(NP)
        goff = g * NP

        def _shift(k, _):
            src_idx[pl.ds(k * 16, 16)] = src_idx[pl.ds(k * 16, 16)] + off
            return _
        lax.fori_loop(0, NCH, _shift, 0)

        def _zden(i, _):
            for v in range(8):
                denom_v[i, pl.ds(v * 16, 16)] = zero16
            return _
        lax.fori_loop(0, NP // 128, _zden, 0)
        pltpu.sync_copy(denom_v.at[pl.ds(s * (NP // 128 // NT), NP // 128 // NT)],
                        denom_s.at[pl.ds(s * (NP // 128 // NT), NP // 128 // NT)])

        pltpu.sync_copy(asrc_hbm.at[g], big_v.at[pl.ds(0, 80)])
        pltpu.sync_copy(adst_hbm.at[g], big_v.at[pl.ds(80, 80)])
        plsc.subcore_barrier()

        def _edges(k, _):
            sv_g = src_idx[pl.ds(k * 16, 16)] - goff
            dv = dst_v[pl.ds(k * 16, 16)]
            ea = plsc.load_gather(big_v, [sv_g // 128, sv_g % 128])
            eb = plsc.load_gather(big_v, [80 + dv // 128, dv % 128])
            e = ea + eb
            e = jnp.maximum(e, 0.2 * e)
            p = jnp.exp(e)
            eid = k * 16 + lax.iota(jnp.int32, 16)
            p = jnp.where(eid < EPT, p, 0.0)
            p_alpha[pl.ds(k * 16, 16)] = p
            plsc.addupdate_scatter(denom_v, [dv // 128, dv % 128], p)
            return _
        lax.fori_loop(0, NCH, _edges, 0)

        pltpu.sync_copy(denom_v, denom_s.at[idx80_v], add=True)
        plsc.subcore_barrier()
        pltpu.sync_copy(denom_s, denom_v)

        def _alpha(k, _):
            dv = dst_v[pl.ds(k * 16, 16)]
            den = plsc.load_gather(denom_v, [dv // 128, dv % 128])
            p = p_alpha[pl.ds(k * 16, 16)]
            p_alpha[pl.ds(k * 16, 16)] = p / (den + 1e-16)
            return _
        lax.fori_loop(0, NCH, _alpha, 0)

        for lo, pr in PASSES:
            def _zrows(i, _):
                for v in range(D // 16):
                    big_v[i, pl.ds(v * 16, 16)] = zero16
                return _
            lax.fori_loop(0, 128, _zrows, 0)
            stripe = pr // NT
            for j in range(stripe // 128):
                pltpu.sync_copy(big_v.at[pl.ds(0, 128)],
                                acc_s.at[pl.ds(s * stripe + j * 128, 128)])
            plsc.subcore_barrier()

            def _rows(k, _):
                cp = pltpu.async_copy(xs_hbm.at[src_idx.at[pl.ds(k * 128, 128)]],
                                      big_v.at[pl.ds(0, 128)], sem)
                for q in range(8):
                    dv16 = dst_v[pl.ds(k * 128 + q * 16, 16)]
                    inr = (dv16 >= lo) & (dv16 < lo + pr)
                    scidx_v[pl.ds(q * 16, 16)] = jnp.where(inr, dv16 - lo,
                                                           ACC_ROWS)
                cp.wait()

                def _scale(j, _2):
                    ab = plsc.load_gather(
                        p_alpha, [lax.broadcast(k * 128 + j, (16,))])
                    for v in range(D // 16):
                        big_v[j, pl.ds(v * 16, 16)] = (
                            big_v[j, pl.ds(v * 16, 16)] * ab)
                    return _2
                lax.fori_loop(0, 128, _scale, 0)
                pltpu.sync_copy(big_v.at[pl.ds(0, 128)], acc_s.at[scidx_v],
                                add=True)
                return _
            lax.fori_loop(0, NITER, _rows, 0)
            plsc.subcore_barrier()

            for j in range(stripe // 128):
                pltpu.sync_copy(acc_s.at[pl.ds(s * stripe + j * 128, 128)],
                                out_hbm.at[g, pl.ds(lo + s * stripe + j * 128, 128)])
            plsc.subcore_barrier()


def _sc_conv(xs_flat, asrc, adst, src_e, dst_e):
    mesh = plsc.VectorSubcoreMesh(core_axis_name="c", subcore_axis_name="s")
    kfn = functools.partial(
        pl.kernel,
        mesh=mesh,
        compiler_params=pltpu.CompilerParams(needs_layout_passes=False),
        out_type=jax.ShapeDtypeStruct((4, NP, D), jnp.float32),
        scratch_types=[
            pltpu.VMEM((EPAD,), jnp.int32),
            pltpu.VMEM((EPAD,), jnp.int32),
            pltpu.VMEM((EPAD,), jnp.float32),
            pltpu.VMEM((160, 128), jnp.float32),
            pltpu.VMEM((NP // 128, 128), jnp.float32),
            pltpu.VMEM((NP // 128,), jnp.int32),
            pltpu.VMEM((128,), jnp.int32),
            pltpu.VMEM_SHARED((ACC_ROWS + 8, D), jnp.float32),
            pltpu.VMEM_SHARED((NP // 128, 128), jnp.float32),
            pltpu.SemaphoreType.DMA,
        ],
    )(_sc_body)
    return kfn(xs_flat, asrc, adst, src_e, dst_e)


def kernel(x, edge_index, h, c, W, b, Wsrc, Wdst, att_src, att_dst, gat_bias):
    pad = ((0, NP - N), (0, 0))
    xp = jnp.pad(x, pad)
    hp = jnp.pad(h, pad)
    cp = jnp.pad(c, pad)

    nblk = 10
    bs = NP // nblk
    xs, asrc, adst, lin = pl.pallas_call(
        _phase1_body,
        grid=(nblk,),
        in_specs=[
            pl.BlockSpec((bs, D), lambda i: (i, 0)),
            pl.BlockSpec((bs, D), lambda i: (i, 0)),
            pl.BlockSpec((4, D, D), lambda i: (0, 0, 0)),
            pl.BlockSpec((4, D), lambda i: (0, 0)),
            pl.BlockSpec((4, D, D), lambda i: (0, 0, 0)),
            pl.BlockSpec((4, D, D), lambda i: (0, 0, 0)),
            pl.BlockSpec((4, D), lambda i: (0, 0)),
            pl.BlockSpec((4, D), lambda i: (0, 0)),
            pl.BlockSpec((4, D), lambda i: (0, 0)),
        ],
        out_specs=(
            pl.BlockSpec((4, bs, D), lambda i: (0, i, 0)),
            pl.BlockSpec((4, bs), lambda i: (0, i)),
            pl.BlockSpec((4, bs), lambda i: (0, i)),
            pl.BlockSpec((4, bs, D), lambda i: (0, i, 0)),
        ),
        out_shape=(
            jax.ShapeDtypeStruct((4, NP, D), jnp.float32),
            jax.ShapeDtypeStruct((4, NP), jnp.float32),
            jax.ShapeDtypeStruct((4, NP), jnp.float32),
            jax.ShapeDtypeStruct((4, NP, D), jnp.float32),
        ),
    )(xp, hp, W, b, Wsrc, Wdst, att_src, att_dst, gat_bias)

    xs_flat = xs.reshape(4 * NP, D)
    conv = _sc_conv(xs_flat, asrc.reshape(4, NP // 128, 128),
                    adst.reshape(4, NP // 128, 128),
                    edge_index[0], edge_index[1])

    h_new, c_new = pl.pallas_call(
        _phase3_body,
        grid=(nblk,),
        in_specs=[
            pl.BlockSpec((4, bs, D), lambda i: (0, i, 0)),
            pl.BlockSpec((4, bs, D), lambda i: (0, i, 0)),
            pl.BlockSpec((bs, D), lambda i: (i, 0)),
        ],
        out_specs=(
            pl.BlockSpec((bs, D), lambda i: (i, 0)),
            pl.BlockSpec((bs, D), lambda i: (i, 0)),
        ),
        out_shape=(
            jax.ShapeDtypeStruct((NP, D), jnp.float32),
            jax.ShapeDtypeStruct((NP, D), jnp.float32),
        ),
    )(lin, conv, cp)
    return h_new[:N], c_new[:N]


# speedup vs baseline: 13.5597x; 1.2641x over previous
import functools

import jax
import jax.numpy as jnp
from jax import lax
from jax.experimental import pallas as pl
from jax.experimental.pallas import tpu as pltpu
from jax.experimental.pallas import tpu_sc as plsc

N = 10000
NP = 10240
D = 128
E = 320000
NT = 16
EPT = E // NT
CH = 96
NCC = 210
EPAD = CH * NCC
NCH = EPAD // 16
ROWS_PER_TILE = NP // NT
ACC_ROWS = 4096
PASSES = ((0, 4096), (4096, 4096), (8192, 2048))


def _phase1_body(x_ref, h_ref, W_ref, b_ref, Wsrc_ref, Wdst_ref, asw_ref,
                 adw_ref, gb_ref, xs_ref, asrc_ref, adst_ref, lin_ref):
    xb = x_ref[...]
    hb = h_ref[...]
    for g in range(4):
        xsg = jnp.dot(hb, Wsrc_ref[g], preferred_element_type=jnp.float32)
        xs_ref[g] = xsg
        asrc_ref[g] = jnp.sum(xsg * asw_ref[g][None, :], axis=1)
        wv = jnp.sum(Wdst_ref[g] * adw_ref[g][None, :], axis=1)
        adst_ref[g] = jnp.sum(hb * wv[None, :], axis=1)
        lin_ref[g] = (jnp.dot(xb, W_ref[g], preferred_element_type=jnp.float32)
                      + (b_ref[g] + gb_ref[g])[None, :])


def _phase3_body(lin_ref, conv_ref, c_ref, h_out, c_out):
    gates = lin_ref[...] + conv_ref[...]
    ig = jax.nn.sigmoid(gates[0])
    fg = jax.nn.sigmoid(gates[1])
    tg = jnp.tanh(gates[2])
    c_new = fg * c_ref[...] + ig * tg
    og = jax.nn.sigmoid(gates[3])
    h_out[...] = og * jnp.tanh(c_new)
    c_out[...] = c_new


def _sc_body(xs_hbm, asrc_hbm, adst_hbm, srce_hbm, dste_hbm, out_hbm,
             src_idx, dst_v, p_alpha, big_v, denom_v, idx80_v,
             scidx_v, acc_s, denom_s, sem_a, sem_b):
    c = lax.axis_index("c")
    s = lax.axis_index("s")
    ebase = s * EPT

    zero16 = jnp.zeros((16,), jnp.float32)
    zero16i = jnp.zeros((16,), jnp.int32)
    for q in range(NP // 128 // 16):
        idx80_v[pl.ds(q * 16, 16)] = lax.iota(jnp.int32, 16) + (q * 16)

    pltpu.sync_copy(srce_hbm.at[pl.ds(ebase, EPT)], src_idx.at[pl.ds(0, EPT)])
    pltpu.sync_copy(dste_hbm.at[pl.ds(ebase, EPT)], dst_v.at[pl.ds(0, EPT)])
    for i in range((EPAD - EPT) // 16):
        src_idx[pl.ds(EPT + i * 16, 16)] = zero16i
        dst_v[pl.ds(EPT + i * 16, 16)] = zero16i

    for gg in range(2):
        g = 2 * c + gg

        off = (c * (2 * NP)).astype(jnp.int32) if gg == 0 else jnp.int32(NP)
        goff = g * NP

        def _shift(k, _):
            src_idx[pl.ds(k * 16, 16)] = src_idx[pl.ds(k * 16, 16)] + off
            return _
        lax.fori_loop(0, NCH, _shift, 0)

        def _zden(i, _):
            for v in range(8):
                denom_v[i, pl.ds(v * 16, 16)] = zero16
            return _
        lax.fori_loop(0, NP // 128, _zden, 0)
        pltpu.sync_copy(denom_v.at[pl.ds(s * (NP // 128 // NT), NP // 128 // NT)],
                        denom_s.at[pl.ds(s * (NP // 128 // NT), NP // 128 // NT)])

        pltpu.sync_copy(asrc_hbm.at[g], big_v.at[pl.ds(0, 80)])
        pltpu.sync_copy(adst_hbm.at[g], big_v.at[pl.ds(80, 80)])
        plsc.subcore_barrier()

        def _edges(k, _):
            sv_g = src_idx[pl.ds(k * 16, 16)] - goff
            dv = dst_v[pl.ds(k * 16, 16)]
            ea = plsc.load_gather(big_v, [sv_g // 128, sv_g % 128])
            eb = plsc.load_gather(big_v, [80 + dv // 128, dv % 128])
            e = ea + eb
            e = jnp.maximum(e, 0.2 * e)
            p = jnp.exp(e)
            eid = k * 16 + lax.iota(jnp.int32, 16)
            p = jnp.where(eid < EPT, p, 0.0)
            p_alpha[pl.ds(k * 16, 16)] = p
            plsc.addupdate_scatter(denom_v, [dv // 128, dv % 128], p)
            return _
        lax.fori_loop(0, NCH, _edges, 0)

        pltpu.sync_copy(denom_v, denom_s.at[idx80_v], add=True)
        plsc.subcore_barrier()
        pltpu.sync_copy(denom_s, denom_v)

        def _alpha(k, _):
            dv = dst_v[pl.ds(k * 16, 16)]
            den = plsc.load_gather(denom_v, [dv // 128, dv % 128])
            p = p_alpha[pl.ds(k * 16, 16)]
            p_alpha[pl.ds(k * 16, 16)] = p / (den + 1e-16)
            return _
        lax.fori_loop(0, NCH, _alpha, 0)

        for lo, pr in PASSES:
            def _zrows(i, _):
                for v in range(D // 16):
                    big_v[i, pl.ds(v * 16, 16)] = zero16
                return _
            lax.fori_loop(0, 128, _zrows, 0)
            stripe = pr // NT
            for j in range(stripe // 128):
                pltpu.sync_copy(big_v.at[pl.ds(0, 128)],
                                acc_s.at[pl.ds(s * stripe + j * 128, 128)])
            plsc.subcore_barrier()

            def _issue(kc, buf, sem):
                pltpu.async_copy(
                    xs_hbm.at[src_idx.at[pl.ds(kc * CH, CH)]],
                    big_v.at[pl.ds(buf * CH, CH)], sem)

            def _drain(sem):
                pltpu.make_async_copy(xs_hbm.at[pl.ds(0, CH)],
                                      big_v.at[pl.ds(0, CH)], sem).wait()

            def _process(k, buf):
                bofs = buf * CH
                for q in range(CH // 16):
                    dv16 = dst_v[pl.ds(k * 16 * (CH // 16) + q * 16, 16)]
                    inr = (dv16 >= lo) & (dv16 < lo + pr)
                    scidx_v[pl.ds(q * 16, 16)] = jnp.where(inr, dv16 - lo,
                                                           ACC_ROWS)

                def _scale(j, _2):
                    ab = plsc.load_gather(
                        p_alpha, [lax.broadcast(k * CH + j, (16,))])
                    for v in range(D // 16):
                        big_v[bofs + j, pl.ds(v * 16, 16)] = (
                            big_v[bofs + j, pl.ds(v * 16, 16)] * ab)
                    return _2
                lax.fori_loop(0, CH, _scale, 0)
                pltpu.sync_copy(big_v.at[pl.ds(bofs, CH)], acc_s.at[scidx_v],
                                add=True)

            _issue(jnp.int32(0), 0, sem_a)

            def _pairs(m, _):
                ka = 2 * m
                kb = 2 * m + 1
                _issue(kb, 1, sem_b)
                _drain(sem_a)
                _process(ka, 0)
                _issue((kb + 1) % NCC, 0, sem_a)
                _drain(sem_b)
                _process(kb, 1)
                return _
            lax.fori_loop(0, NCC // 2, _pairs, 0)
            _drain(sem_a)
            plsc.subcore_barrier()

            for j in range(stripe // 128):
                pltpu.sync_copy(acc_s.at[pl.ds(s * stripe + j * 128, 128)],
                                out_hbm.at[g, pl.ds(lo + s * stripe + j * 128, 128)])
            plsc.subcore_barrier()


def _sc_conv(xs_flat, asrc, adst, src_e, dst_e):
    mesh = plsc.VectorSubcoreMesh(core_axis_name="c", subcore_axis_name="s")
    kfn = functools.partial(
        pl.kernel,
        mesh=mesh,
        compiler_params=pltpu.CompilerParams(needs_layout_passes=False),
        out_type=jax.ShapeDtypeStruct((4, NP, D), jnp.float32),
        scratch_types=[
            pltpu.VMEM((EPAD,), jnp.int32),
            pltpu.VMEM((EPAD,), jnp.int32),
            pltpu.VMEM((EPAD,), jnp.float32),
            pltpu.VMEM((192, 128), jnp.float32),
            pltpu.VMEM((NP // 128, 128), jnp.float32),
            pltpu.VMEM((NP // 128,), jnp.int32),
            pltpu.VMEM((CH,), jnp.int32),
            pltpu.VMEM_SHARED((ACC_ROWS + 8, D), jnp.float32),
            pltpu.VMEM_SHARED((NP // 128, 128), jnp.float32),
            pltpu.SemaphoreType.DMA,
            pltpu.SemaphoreType.DMA,
        ],
    )(_sc_body)
    return kfn(xs_flat, asrc, adst, src_e, dst_e)


def kernel(x, edge_index, h, c, W, b, Wsrc, Wdst, att_src, att_dst, gat_bias):
    pad = ((0, NP - N), (0, 0))
    xp = jnp.pad(x, pad)
    hp = jnp.pad(h, pad)
    cp = jnp.pad(c, pad)

    nblk = 10
    bs = NP // nblk
    xs, asrc, adst, lin = pl.pallas_call(
        _phase1_body,
        grid=(nblk,),
        in_specs=[
            pl.BlockSpec((bs, D), lambda i: (i, 0)),
            pl.BlockSpec((bs, D), lambda i: (i, 0)),
            pl.BlockSpec((4, D, D), lambda i: (0, 0, 0)),
            pl.BlockSpec((4, D), lambda i: (0, 0)),
            pl.BlockSpec((4, D, D), lambda i: (0, 0, 0)),
            pl.BlockSpec((4, D, D), lambda i: (0, 0, 0)),
            pl.BlockSpec((4, D), lambda i: (0, 0)),
            pl.BlockSpec((4, D), lambda i: (0, 0)),
            pl.BlockSpec((4, D), lambda i: (0, 0)),
        ],
        out_specs=(
            pl.BlockSpec((4, bs, D), lambda i: (0, i, 0)),
            pl.BlockSpec((4, bs), lambda i: (0, i)),
            pl.BlockSpec((4, bs), lambda i: (0, i)),
            pl.BlockSpec((4, bs, D), lambda i: (0, i, 0)),
        ),
        out_shape=(
            jax.ShapeDtypeStruct((4, NP, D), jnp.float32),
            jax.ShapeDtypeStruct((4, NP), jnp.float32),
            jax.ShapeDtypeStruct((4, NP), jnp.float32),
            jax.ShapeDtypeStruct((4, NP, D), jnp.float32),
        ),
    )(xp, hp, W, b, Wsrc, Wdst, att_src, att_dst, gat_bias)

    xs_flat = xs.reshape(4 * NP, D)
    conv = _sc_conv(xs_flat, asrc.reshape(4, NP // 128, 128),
                    adst.reshape(4, NP // 128, 128),
                    edge_index[0], edge_index[1])

    h_new, c_new = pl.pallas_call(
        _phase3_body,
        grid=(nblk,),
        in_specs=[
            pl.BlockSpec((4, bs, D), lambda i: (0, i, 0)),
            pl.BlockSpec((4, bs, D), lambda i: (0, i, 0)),
            pl.BlockSpec((bs, D), lambda i: (i, 0)),
        ],
        out_specs=(
            pl.BlockSpec((bs, D), lambda i: (i, 0)),
            pl.BlockSpec((bs, D), lambda i: (i, 0)),
        ),
        out_shape=(
            jax.ShapeDtypeStruct((NP, D), jnp.float32),
            jax.ShapeDtypeStruct((NP, D), jnp.float32),
        ),
    )(lin, conv, cp)
    return h_new[:N], c_new[:N]


# speedup vs baseline: 14.3272x; 1.0566x over previous
import functools

import jax
import jax.numpy as jnp
from jax import lax
from jax.experimental import pallas as pl
from jax.experimental.pallas import tpu as pltpu
from jax.experimental.pallas import tpu_sc as plsc

N = 10000
NP = 10240
D = 128
E = 320000
NT = 16
EPT = E // NT
CH = 64
NCC = 315
EPAD = CH * NCC
NCH = EPAD // 16
ROWS_PER_TILE = NP // NT
ACC_ROWS = 4096
PASSES = ((0, 4096), (4096, 4096), (8192, 2048))


def _phase1_body(x_ref, h_ref, W_ref, b_ref, Wsrc_ref, Wdst_ref, asw_ref,
                 adw_ref, gb_ref, xs_ref, asrc_ref, adst_ref, lin_ref):
    xb = x_ref[...]
    hb = h_ref[...]
    for g in range(4):
        xsg = jnp.dot(hb, Wsrc_ref[g], preferred_element_type=jnp.float32)
        xs_ref[g] = xsg
        asrc_ref[g] = jnp.sum(xsg * asw_ref[g][None, :], axis=1)
        wv = jnp.sum(Wdst_ref[g] * adw_ref[g][None, :], axis=1)
        adst_ref[g] = jnp.sum(hb * wv[None, :], axis=1)
        lin_ref[g] = (jnp.dot(xb, W_ref[g], preferred_element_type=jnp.float32)
                      + (b_ref[g] + gb_ref[g])[None, :])


def _phase3_body(lin_ref, conv_ref, c_ref, h_out, c_out):
    gates = lin_ref[...] + conv_ref[...]
    ig = jax.nn.sigmoid(gates[0])
    fg = jax.nn.sigmoid(gates[1])
    tg = jnp.tanh(gates[2])
    c_new = fg * c_ref[...] + ig * tg
    og = jax.nn.sigmoid(gates[3])
    h_out[...] = og * jnp.tanh(c_new)
    c_out[...] = c_new


def _sc_body(xs_hbm, asrc_hbm, adst_hbm, srce_hbm, dste_hbm, out_hbm,
             src_idx, dst_v, p_alpha, big_v, denom_v, idx80_v,
             scidx_v, acc_s, denom_s, gsem0, gsem1, gsem2, ssem0, ssem1,
             ssem2):
    c = lax.axis_index("c")
    s = lax.axis_index("s")
    ebase = s * EPT

    zero16 = jnp.zeros((16,), jnp.float32)
    zero16i = jnp.zeros((16,), jnp.int32)
    for q in range(NP // 128 // 16):
        idx80_v[pl.ds(q * 16, 16)] = lax.iota(jnp.int32, 16) + (q * 16)

    pltpu.sync_copy(srce_hbm.at[pl.ds(ebase, EPT)], src_idx.at[pl.ds(0, EPT)])
    pltpu.sync_copy(dste_hbm.at[pl.ds(ebase, EPT)], dst_v.at[pl.ds(0, EPT)])
    for i in range((EPAD - EPT) // 16):
        src_idx[pl.ds(EPT + i * 16, 16)] = zero16i
        dst_v[pl.ds(EPT + i * 16, 16)] = zero16i

    for gg in range(2):
        g = 2 * c + gg

        off = (c * (2 * NP)).astype(jnp.int32) if gg == 0 else jnp.int32(NP)
        goff = g * NP

        def _shift(k, _):
            src_idx[pl.ds(k * 16, 16)] = src_idx[pl.ds(k * 16, 16)] + off
            return _
        lax.fori_loop(0, NCH, _shift, 0)

        def _zden(i, _):
            for v in range(8):
                denom_v[i, pl.ds(v * 16, 16)] = zero16
            return _
        lax.fori_loop(0, NP // 128, _zden, 0)
        pltpu.sync_copy(denom_v.at[pl.ds(s * (NP // 128 // NT), NP // 128 // NT)],
                        denom_s.at[pl.ds(s * (NP // 128 // NT), NP // 128 // NT)])

        pltpu.sync_copy(asrc_hbm.at[g], big_v.at[pl.ds(0, 80)])
        pltpu.sync_copy(adst_hbm.at[g], big_v.at[pl.ds(80, 80)])
        plsc.subcore_barrier()

        def _edges(k, _):
            sv_g = src_idx[pl.ds(k * 16, 16)] - goff
            dv = dst_v[pl.ds(k * 16, 16)]
            ea = plsc.load_gather(big_v, [sv_g // 128, sv_g % 128])
            eb = plsc.load_gather(big_v, [80 + dv // 128, dv % 128])
            e = ea + eb
            e = jnp.maximum(e, 0.2 * e)
            p = jnp.exp(e)
            eid = k * 16 + lax.iota(jnp.int32, 16)
            p = jnp.where(eid < EPT, p, 0.0)
            p_alpha[pl.ds(k * 16, 16)] = p
            plsc.addupdate_scatter(denom_v, [dv // 128, dv % 128], p)
            return _
        lax.fori_loop(0, NCH, _edges, 0)

        pltpu.sync_copy(denom_v, denom_s.at[idx80_v], add=True)
        plsc.subcore_barrier()
        pltpu.sync_copy(denom_s, denom_v)

        def _alpha(k, _):
            dv = dst_v[pl.ds(k * 16, 16)]
            den = plsc.load_gather(denom_v, [dv // 128, dv % 128])
            p = p_alpha[pl.ds(k * 16, 16)]
            p_alpha[pl.ds(k * 16, 16)] = p / (den + 1e-16)
            return _
        lax.fori_loop(0, NCH, _alpha, 0)

        for lo, pr in PASSES:
            def _zrows(i, _):
                for v in range(D // 16):
                    big_v[i, pl.ds(v * 16, 16)] = zero16
                return _
            lax.fori_loop(0, 128, _zrows, 0)
            stripe = pr // NT
            for j in range(stripe // 128):
                pltpu.sync_copy(big_v.at[pl.ds(0, 128)],
                                acc_s.at[pl.ds(s * stripe + j * 128, 128)])
            plsc.subcore_barrier()

            gsem = (gsem0, gsem1, gsem2)
            ssem = (ssem0, ssem1, ssem2)

            def _issue(kc, buf, sem):
                pltpu.async_copy(
                    xs_hbm.at[src_idx.at[pl.ds(kc * CH, CH)]],
                    big_v.at[pl.ds(buf * CH, CH)], sem)

            def _drain(sem):
                pltpu.make_async_copy(xs_hbm.at[pl.ds(0, CH)],
                                      big_v.at[pl.ds(0, CH)], sem).wait()

            def _chunk(k, t):
                bofs = t * CH
                _drain(gsem[t])
                for q in range(CH // 16):
                    dv16 = dst_v[pl.ds(k * CH + q * 16, 16)]
                    inr = (dv16 >= lo) & (dv16 < lo + pr)
                    scidx_v[t, pl.ds(q * 16, 16)] = jnp.where(inr, dv16 - lo,
                                                              ACC_ROWS)

                def _scale(j, _2):
                    ab = plsc.load_gather(
                        p_alpha, [lax.broadcast(k * CH + j, (16,))])
                    for v in range(D // 16):
                        big_v[bofs + j, pl.ds(v * 16, 16)] = (
                            big_v[bofs + j, pl.ds(v * 16, 16)] * ab)
                    return _2
                lax.fori_loop(0, CH, _scale, 0)
                pltpu.async_copy(big_v.at[pl.ds(bofs, CH)],
                                 acc_s.at[scidx_v.at[t]], ssem[t], add=True)
                t2 = (t + 2) % 3

                @pl.when(k > 0)
                def _():
                    _drain(ssem[t2])

                @pl.when(k + 2 < NCC)
                def _():
                    _issue(k + 2, t2, gsem[t2])

            _issue(jnp.int32(0), 0, gsem0)
            _issue(jnp.int32(1), 1, gsem1)

            def _triples(m, _):
                _chunk(3 * m, 0)
                _chunk(3 * m + 1, 1)
                _chunk(3 * m + 2, 2)
                return _
            lax.fori_loop(0, NCC // 3, _triples, 0)
            _drain(ssem[(NCC - 1) % 3])
            plsc.subcore_barrier()

            for j in range(stripe // 128):
                pltpu.sync_copy(acc_s.at[pl.ds(s * stripe + j * 128, 128)],
                                out_hbm.at[g, pl.ds(lo + s * stripe + j * 128, 128)])
            plsc.subcore_barrier()


def _sc_conv(xs_flat, asrc, adst, src_e, dst_e):
    mesh = plsc.VectorSubcoreMesh(core_axis_name="c", subcore_axis_name="s")
    kfn = functools.partial(
        pl.kernel,
        mesh=mesh,
        compiler_params=pltpu.CompilerParams(needs_layout_passes=False),
        out_type=jax.ShapeDtypeStruct((4, NP, D), jnp.float32),
        scratch_types=[
            pltpu.VMEM((EPAD,), jnp.int32),
            pltpu.VMEM((EPAD,), jnp.int32),
            pltpu.VMEM((EPAD,), jnp.float32),
            pltpu.VMEM((192, 128), jnp.float32),
            pltpu.VMEM((NP // 128, 128), jnp.float32),
            pltpu.VMEM((NP // 128,), jnp.int32),
            pltpu.VMEM((3, CH), jnp.int32),
            pltpu.VMEM_SHARED((ACC_ROWS + 8, D), jnp.float32),
            pltpu.VMEM_SHARED((NP // 128, 128), jnp.float32),
            pltpu.SemaphoreType.DMA,
            pltpu.SemaphoreType.DMA,
            pltpu.SemaphoreType.DMA,
            pltpu.SemaphoreType.DMA,
            pltpu.SemaphoreType.DMA,
            pltpu.SemaphoreType.DMA,
        ],
    )(_sc_body)
    return kfn(xs_flat, asrc, adst, src_e, dst_e)


def kernel(x, edge_index, h, c, W, b, Wsrc, Wdst, att_src, att_dst, gat_bias):
    pad = ((0, NP - N), (0, 0))
    xp = jnp.pad(x, pad)
    hp = jnp.pad(h, pad)
    cp = jnp.pad(c, pad)

    nblk = 10
    bs = NP // nblk
    xs, asrc, adst, lin = pl.pallas_call(
        _phase1_body,
        grid=(nblk,),
        in_specs=[
            pl.BlockSpec((bs, D), lambda i: (i, 0)),
            pl.BlockSpec((bs, D), lambda i: (i, 0)),
            pl.BlockSpec((4, D, D), lambda i: (0, 0, 0)),
            pl.BlockSpec((4, D), lambda i: (0, 0)),
            pl.BlockSpec((4, D, D), lambda i: (0, 0, 0)),
            pl.BlockSpec((4, D, D), lambda i: (0, 0, 0)),
            pl.BlockSpec((4, D), lambda i: (0, 0)),
            pl.BlockSpec((4, D), lambda i: (0, 0)),
            pl.BlockSpec((4, D), lambda i: (0, 0)),
        ],
        out_specs=(
            pl.BlockSpec((4, bs, D), lambda i: (0, i, 0)),
            pl.BlockSpec((4, bs), lambda i: (0, i)),
            pl.BlockSpec((4, bs), lambda i: (0, i)),
            pl.BlockSpec((4, bs, D), lambda i: (0, i, 0)),
        ),
        out_shape=(
            jax.ShapeDtypeStruct((4, NP, D), jnp.float32),
            jax.ShapeDtypeStruct((4, NP), jnp.float32),
            jax.ShapeDtypeStruct((4, NP), jnp.float32),
            jax.ShapeDtypeStruct((4, NP, D), jnp.float32),
        ),
    )(xp, hp, W, b, Wsrc, Wdst, att_src, att_dst, gat_bias)

    xs_flat = xs.reshape(4 * NP, D)
    conv = _sc_conv(xs_flat, asrc.reshape(4, NP // 128, 128),
                    adst.reshape(4, NP // 128, 128),
                    edge_index[0], edge_index[1])

    h_new, c_new = pl.pallas_call(
        _phase3_body,
        grid=(nblk,),
        in_specs=[
            pl.BlockSpec((4, bs, D), lambda i: (0, i, 0)),
            pl.BlockSpec((4, bs, D), lambda i: (0, i, 0)),
            pl.BlockSpec((bs, D), lambda i: (i, 0)),
        ],
        out_specs=(
            pl.BlockSpec((bs, D), lambda i: (i, 0)),
            pl.BlockSpec((bs, D), lambda i: (i, 0)),
        ),
        out_shape=(
            jax.ShapeDtypeStruct((NP, D), jnp.float32),
            jax.ShapeDtypeStruct((NP, D), jnp.float32),
        ),
    )(lin, conv, cp)
    return h_new[:N], c_new[:N]
